# baseline (speedup 1.0000x reference)
import jax
import jax.numpy as jnp
from jax import lax
from jax.experimental import pallas as pl
from jax.experimental.pallas import tpu as pltpu
from jax.experimental.pallas import tpu_sc as plsc

U = 128
F = 100000
B = 16384
NC, NS, L = 2, 16, 16
NW = NC * NS
E = B // NW
G = 4
NCHUNK = E // G
W = G * U * 2
PER_ELEM = U * 2
LN10 = 2.302585092994046


def _body(mu_hbm, ls_hbm, cmu_hbm, cls_hbm, fids_hbm, epsc_hbm, epse_hbm,
          out_e_hbm, out_c_hbm,
          fids_v, epsc_v, cmu_v, cls_v, outc_v,
          ufc_v, idx_v, mu_v, ls_v, eps_v, out_v, sem):
  wid = lax.axis_index("s") * NC + lax.axis_index("c")
  base = wid * E

  pltpu.sync_copy(fids_hbm.at[pl.ds(base, E)], fids_v)
  pltpu.sync_copy(epsc_hbm.at[pl.ds(base, E)], epsc_v)

  handles = []
  for c in range(E // 128):
    s = pl.ds(c * 128, 128)
    handles.append(pltpu.async_copy(cmu_hbm.at[fids_v.at[s]], cmu_v.at[s], sem))
    handles.append(pltpu.async_copy(cls_hbm.at[fids_v.at[s]], cls_v.at[s], sem))
  for h in handles:
    h.wait()

  @pl.loop(0, E // L)
  def _conc(i):
    s = pl.ds(i * L, L)
    v = cmu_v[s] + jnp.exp(cls_v[s]) * epsc_v[s]
    outc_v[s] = jnp.exp(LN10 * v)

  pltpu.sync_copy(outc_v, out_c_hbm.at[pl.ds(base, E)])

  for s16 in range(PER_ELEM // L):
    p = lax.iota(jnp.int32, L) + (s16 * L)
    ufc_v[pl.ds(s16 * L, L)] = (p >> 1) * (2 * F) + (p & 1)

  @pl.loop(0, NCHUNK)
  def _chunk(t):
    e0 = t * G
    g0 = base + e0
    pltpu.sync_copy(epse_hbm.at[pl.ds(g0 * PER_ELEM, W)], eps_v)
    for g in range(G):
      f_spl = plsc.load_gather(fids_v, [jnp.full((L,), e0 + g, jnp.int32)])
      f2 = f_spl + f_spl
      for s16 in range(PER_ELEM // L):
        o = g * PER_ELEM + s16 * L
        idx_v[pl.ds(o, L)] = ufc_v[pl.ds(s16 * L, L)] + f2
    hs = []
    for st in range(W // 128):
      s = pl.ds(st * 128, 128)
      hs.append(pltpu.async_copy(mu_hbm.at[idx_v.at[s]], mu_v.at[s], sem))
      hs.append(pltpu.async_copy(ls_hbm.at[idx_v.at[s]], ls_v.at[s], sem))
    for h in hs:
      h.wait()

    @pl.loop(0, W // L)
    def _cmp(i):
      s = pl.ds(i * L, L)
      out_v[s] = mu_v[s] + jnp.exp(ls_v[s]) * eps_v[s]

    pltpu.sync_copy(out_v, out_e_hbm.at[pl.ds(g0 * PER_ELEM, W)])


@jax.jit
def _run(mu_flat, ls_flat, cmu, cls, fids, epsc, epse_flat):
  mesh = plsc.VectorSubcoreMesh(core_axis_name="c", subcore_axis_name="s")
  return pl.kernel(
      _body,
      out_type=(
          jax.ShapeDtypeStruct((B * U * 2,), jnp.float32),
          jax.ShapeDtypeStruct((B,), jnp.float32),
      ),
      mesh=mesh,
      compiler_params=pltpu.CompilerParams(
          needs_layout_passes=False, use_tc_tiling_on_sc=False),
      scratch_types=[
          pltpu.VMEM((E,), jnp.int32),
          pltpu.VMEM((E,), jnp.float32),
          pltpu.VMEM((E,), jnp.float32),
          pltpu.VMEM((E,), jnp.float32),
          pltpu.VMEM((E,), jnp.float32),
          pltpu.VMEM((PER_ELEM,), jnp.int32),
          pltpu.VMEM((W,), jnp.int32),
          pltpu.VMEM((W,), jnp.float32),
          pltpu.VMEM((W,), jnp.float32),
          pltpu.VMEM((W,), jnp.float32),
          pltpu.VMEM((W,), jnp.float32),
          pltpu.SemaphoreType.DMA,
      ],
  )(mu_flat, ls_flat, cmu, cls, fids, epsc, epse_flat)


def kernel(interaction_mu, interaction_log_sigma, conc_mu, conc_log_sigma,
           family_ids, eps_conc, eps_energy, batch_size):
  mu_flat = interaction_mu.reshape(U * F * 2)
  ls_flat = interaction_log_sigma.reshape(U * F * 2)
  epse_flat = eps_energy.reshape(B * U * 2)
  fids = family_ids.astype(jnp.int32)
  out_e, out_c = _run(mu_flat, ls_flat, conc_mu, conc_log_sigma, fids,
                      eps_conc, epse_flat)
  return out_e.reshape(B, U, 2), out_c, family_ids


# speedup vs baseline: 7.5954x; 7.5954x over previous
import jax
import jax.numpy as jnp
from jax import lax
from jax.experimental import pallas as pl
from jax.experimental.pallas import tpu as pltpu
from jax.experimental.pallas import tpu_sc as plsc

U = 128
F = 100000
B = 16384
NC, NS, L = 2, 16, 16
NW = NC * NS
E = B // NW
G = 64
NCHUNK = E // G
W = G * U * 2
PER_ELEM = U * 2
TO, TI, TB = 200, 1000, 8
LN10 = 2.302585092994046


def _tbody(x_ref, o_ref):
  x = x_ref[...].reshape(U, TB * TI)
  o_ref[...] = x.T.reshape(TB, TI, U)


def _transpose(x3d):
  return pl.pallas_call(
      _tbody,
      grid=(TO // TB,),
      in_specs=[pl.BlockSpec((U, TB, TI), lambda k: (0, k, 0))],
      out_specs=pl.BlockSpec((TB, TI, U), lambda k: (k, 0, 0)),
      out_shape=jax.ShapeDtypeStruct((TO, TI, U), jnp.float32),
  )(x3d)


def _body(tmu_hbm, tls_hbm, cmu_hbm, cls_hbm, fids_hbm, epsc_hbm, epse_hbm,
          out_e_hbm, out_c_hbm,
          fids_v, epsc_v, cmu_v, cls_v, outc_v,
          idx_v, mu_v, ls_v, eps_v, out_v, sem):
  wid = lax.axis_index("s") * NC + lax.axis_index("c")
  base = wid * E

  pltpu.sync_copy(fids_hbm.at[pl.ds(base, E)], fids_v)
  pltpu.sync_copy(epsc_hbm.at[pl.ds(base, E)], epsc_v)

  handles = []
  for c in range(E // 128):
    s = pl.ds(c * 128, 128)
    handles.append(pltpu.async_copy(cmu_hbm.at[fids_v.at[s]], cmu_v.at[s], sem))
    handles.append(pltpu.async_copy(cls_hbm.at[fids_v.at[s]], cls_v.at[s], sem))
  for h in handles:
    h.wait()

  @pl.loop(0, E // L)
  def _conc(i):
    s = pl.ds(i * L, L)
    v = cmu_v[s] + jnp.exp(cls_v[s]) * epsc_v[s]
    outc_v[s] = jnp.exp(LN10 * v)

  pltpu.sync_copy(outc_v, out_c_hbm.at[pl.ds(base, E)])

  io2 = lax.iota(jnp.int32, L) * 2
  rh = lax.iota(jnp.int32, L) >> 1
  ch = lax.iota(jnp.int32, L) & 1

  @pl.loop(0, NCHUNK)
  def _chunk(t):
    e0 = t * G
    g0 = base + e0
    pltpu.sync_copy(epse_hbm.at[pl.ds(g0 * PER_ELEM, W)], eps_v)
    for j in range(G // L):
      fvec = fids_v[pl.ds(e0 + j * L, L)]
      f2 = fvec + fvec
      o = 2 * j * L
      plsc.store_scatter(idx_v, [io2 + o], f2)
      plsc.store_scatter(idx_v, [io2 + (o + 1)], f2 + 1)
    h1 = pltpu.async_copy(tmu_hbm.at[idx_v], mu_v, sem)
    h2 = pltpu.async_copy(tls_hbm.at[idx_v], ls_v, sem)
    h1.wait()
    h2.wait()

    @pl.loop(0, G)
    def _elem(e):
      rowb = ch + 2 * e
      eb = e * PER_ELEM
      for i in range(PER_ELEM // L):
        cols = rh + i * (L // 2)
        m16 = plsc.load_gather(mu_v, [rowb, cols])
        l16 = plsc.load_gather(ls_v, [rowb, cols])
        s = pl.ds(eb + i * L, L)
        out_v[s] = m16 + jnp.exp(l16) * eps_v[s]

    pltpu.sync_copy(out_v, out_e_hbm.at[pl.ds(g0 * PER_ELEM, W)])


@jax.jit
def _run(mu3d, ls3d, cmu, cls, fids, epsc, epse_flat):
  tmu = _transpose(mu3d).reshape(2 * F, U)
  tls = _transpose(ls3d).reshape(2 * F, U)
  mesh = plsc.VectorSubcoreMesh(core_axis_name="c", subcore_axis_name="s")
  return pl.kernel(
      _body,
      out_type=(
          jax.ShapeDtypeStruct((B * U * 2,), jnp.float32),
          jax.ShapeDtypeStruct((B,), jnp.float32),
      ),
      mesh=mesh,
      compiler_params=pltpu.CompilerParams(needs_layout_passes=False),
      scratch_types=[
          pltpu.VMEM((E,), jnp.int32),
          pltpu.VMEM((E,), jnp.float32),
          pltpu.VMEM((E,), jnp.float32),
          pltpu.VMEM((E,), jnp.float32),
          pltpu.VMEM((E,), jnp.float32),
          pltpu.VMEM((2 * G,), jnp.int32),
          pltpu.VMEM((2 * G, U), jnp.float32),
          pltpu.VMEM((2 * G, U), jnp.float32),
          pltpu.VMEM((W,), jnp.float32),
          pltpu.VMEM((W,), jnp.float32),
          pltpu.SemaphoreType.DMA,
      ],
  )(tmu, tls, cmu, cls, fids, epsc, epse_flat)


def kernel(interaction_mu, interaction_log_sigma, conc_mu, conc_log_sigma,
           family_ids, eps_conc, eps_energy, batch_size):
  mu3d = interaction_mu.reshape(U, TO, TI)
  ls3d = interaction_log_sigma.reshape(U, TO, TI)
  epse_flat = eps_energy.reshape(B * U * 2)
  fids = family_ids.astype(jnp.int32)
  out_e, out_c = _run(mu3d, ls3d, conc_mu, conc_log_sigma, fids,
                      eps_conc, epse_flat)
  return out_e.reshape(B, U, 2), out_c, family_ids


# speedup vs baseline: 7.6126x; 1.0023x over previous
import jax
import jax.numpy as jnp
from jax import lax
from jax.experimental import pallas as pl
from jax.experimental.pallas import tpu as pltpu
from jax.experimental.pallas import tpu_sc as plsc

U = 128
F = 100000
B = 16384
NC, NS, L = 2, 16, 16
NW = NC * NS
E = B // NW
G = 64
NCHUNK = E // G
W = G * U * 2
PER_ELEM = U * 2
TB = 1664
LN10 = 2.302585092994046


def _tbody(x_ref, o_ref):
  o_ref[...] = x_ref[...].T


def _transpose(x2d):
  return pl.pallas_call(
      _tbody,
      grid=(pl.cdiv(2 * F, TB),),
      in_specs=[pl.BlockSpec((U, TB), lambda k: (0, k))],
      out_specs=pl.BlockSpec((TB, U), lambda k: (k, 0)),
      out_shape=jax.ShapeDtypeStruct((2 * F, U), jnp.float32),
  )(x2d)


def _body(tmu_hbm, tls_hbm, cmu_hbm, cls_hbm, fids_hbm, epsc_hbm, epse_hbm,
          out_e_hbm, out_c_hbm,
          fids_v, epsc_v, cmu_v, cls_v, outc_v,
          idx_v, mu_v, ls_v, eps_v, out_v, sem):
  wid = lax.axis_index("s") * NC + lax.axis_index("c")
  base = wid * E

  pltpu.sync_copy(fids_hbm.at[pl.ds(base, E)], fids_v)
  pltpu.sync_copy(epsc_hbm.at[pl.ds(base, E)], epsc_v)

  handles = []
  for c in range(E // 128):
    s = pl.ds(c * 128, 128)
    handles.append(pltpu.async_copy(cmu_hbm.at[fids_v.at[s]], cmu_v.at[s], sem))
    handles.append(pltpu.async_copy(cls_hbm.at[fids_v.at[s]], cls_v.at[s], sem))
  for h in handles:
    h.wait()

  @pl.loop(0, E // L)
  def _conc(i):
    s = pl.ds(i * L, L)
    v = cmu_v[s] + jnp.exp(cls_v[s]) * epsc_v[s]
    outc_v[s] = jnp.exp(LN10 * v)

  pltpu.sync_copy(outc_v, out_c_hbm.at[pl.ds(base, E)])

  io2 = lax.iota(jnp.int32, L) * 2
  rh = lax.iota(jnp.int32, L) >> 1
  ch = lax.iota(jnp.int32, L) & 1

  @pl.loop(0, NCHUNK)
  def _chunk(t):
    e0 = t * G
    g0 = base + e0
    pltpu.sync_copy(epse_hbm.at[pl.ds(g0 * PER_ELEM, W)], eps_v)
    for j in range(G // L):
      fvec = fids_v[pl.ds(e0 + j * L, L)]
      f2 = fvec + fvec
      o = 2 * j * L
      plsc.store_scatter(idx_v, [io2 + o], f2)
      plsc.store_scatter(idx_v, [io2 + (o + 1)], f2 + 1)
    h1 = pltpu.async_copy(tmu_hbm.at[idx_v], mu_v, sem)
    h2 = pltpu.async_copy(tls_hbm.at[idx_v], ls_v, sem)
    h1.wait()
    h2.wait()

    @pl.loop(0, G)
    def _elem(e):
      rowb = ch + 2 * e
      eb = e * PER_ELEM
      for i in range(PER_ELEM // L):
        cols = rh + i * (L // 2)
        m16 = plsc.load_gather(mu_v, [rowb, cols])
        l16 = plsc.load_gather(ls_v, [rowb, cols])
        s = pl.ds(eb + i * L, L)
        out_v[s] = m16 + jnp.exp(l16) * eps_v[s]

    pltpu.sync_copy(out_v, out_e_hbm.at[pl.ds(g0 * PER_ELEM, W)])


@jax.jit
def _run(mu2d, ls2d, cmu, cls, fids, epsc, epse_flat):
  tmu = _transpose(mu2d)
  tls = _transpose(ls2d)
  mesh = plsc.VectorSubcoreMesh(core_axis_name="c", subcore_axis_name="s")
  return pl.kernel(
      _body,
      out_type=(
          jax.ShapeDtypeStruct((B * U * 2,), jnp.float32),
          jax.ShapeDtypeStruct((B,), jnp.float32),
      ),
      mesh=mesh,
      compiler_params=pltpu.CompilerParams(needs_layout_passes=False),
      scratch_types=[
          pltpu.VMEM((E,), jnp.int32),
          pltpu.VMEM((E,), jnp.float32),
          pltpu.VMEM((E,), jnp.float32),
          pltpu.VMEM((E,), jnp.float32),
          pltpu.VMEM((E,), jnp.float32),
          pltpu.VMEM((2 * G,), jnp.int32),
          pltpu.VMEM((2 * G, U), jnp.float32),
          pltpu.VMEM((2 * G, U), jnp.float32),
          pltpu.VMEM((W,), jnp.float32),
          pltpu.VMEM((W,), jnp.float32),
          pltpu.SemaphoreType.DMA,
      ],
  )(tmu, tls, cmu, cls, fids, epsc, epse_flat)


def kernel(interaction_mu, interaction_log_sigma, conc_mu, conc_log_sigma,
           family_ids, eps_conc, eps_energy, batch_size):
  mu2d = interaction_mu.reshape(U, F * 2)
  ls2d = interaction_log_sigma.reshape(U, F * 2)
  epse_flat = eps_energy.reshape(B * U * 2)
  fids = family_ids.astype(jnp.int32)
  out_e, out_c = _run(mu2d, ls2d, conc_mu, conc_log_sigma, fids,
                      eps_conc, epse_flat)
  return out_e.reshape(B, U, 2), out_c, family_ids


# speedup vs baseline: 49.3675x; 6.4849x over previous
import jax
import jax.numpy as jnp
from jax import lax
from jax.experimental import pallas as pl
from jax.experimental.pallas import tpu as pltpu
from jax.experimental.pallas import tpu_sc as plsc

U = 128
F = 100000
B = 16384
NC, NS, L = 2, 16, 16
NW = NC * NS
E = B // NW
G = 64
NCHUNK = E // G
W = G * U * 2
PER_ELEM = U * 2
TB = 1664
LN10 = 2.302585092994046


def _tbody(x_ref, o_ref):
  o_ref[...] = x_ref[...].T


def _transpose(x2d):
  return pl.pallas_call(
      _tbody,
      grid=(pl.cdiv(2 * F, TB),),
      in_specs=[pl.BlockSpec((U, TB), lambda k: (0, k))],
      out_specs=pl.BlockSpec((TB, U), lambda k: (k, 0)),
      out_shape=jax.ShapeDtypeStruct((2 * F, U), jnp.float32),
  )(x2d)


def _body(tmu_hbm, tls_hbm, cmu_hbm, cls_hbm, fids_hbm, epsc_hbm, epse_hbm,
          out_e_hbm, out_c_hbm,
          fids_v, epsc_v, cmu_v, cls_v, outc_v,
          idx_v, mu_v, ls_v, eps_v, out_v, sem):
  wid = lax.axis_index("s") * NC + lax.axis_index("c")
  base = wid * E

  pltpu.sync_copy(fids_hbm.at[pl.ds(base, E)], fids_v)
  pltpu.sync_copy(epsc_hbm.at[pl.ds(base, E)], epsc_v)

  handles = []
  for c in range(E // 128):
    s = pl.ds(c * 128, 128)
    handles.append(pltpu.async_copy(cmu_hbm.at[fids_v.at[s]], cmu_v.at[s], sem))
    handles.append(pltpu.async_copy(cls_hbm.at[fids_v.at[s]], cls_v.at[s], sem))
  for h in handles:
    h.wait()

  @pl.loop(0, E // L)
  def _conc(i):
    s = pl.ds(i * L, L)
    v = cmu_v[s] + jnp.exp(cls_v[s]) * epsc_v[s]
    outc_v[s] = jnp.exp(LN10 * v)

  pltpu.sync_copy(outc_v, out_c_hbm.at[pl.ds(base, E)])

  io2 = lax.iota(jnp.int32, L) * 2
  rh = lax.iota(jnp.int32, L) >> 1
  ch = lax.iota(jnp.int32, L) & 1

  @pl.loop(0, NCHUNK)
  def _chunk(t):
    e0 = t * G
    g0 = base + e0
    pltpu.sync_copy(epse_hbm.at[pl.ds(g0, G)], eps_v)
    for j in range(G // L):
      fvec = fids_v[pl.ds(e0 + j * L, L)]
      f2 = fvec + fvec
      o = 2 * j * L
      plsc.store_scatter(idx_v, [io2 + o], f2)
      plsc.store_scatter(idx_v, [io2 + (o + 1)], f2 + 1)
    h1 = pltpu.async_copy(tmu_hbm.at[idx_v], mu_v, sem)
    h2 = pltpu.async_copy(tls_hbm.at[idx_v], ls_v, sem)
    h1.wait()
    h2.wait()

    @pl.loop(0, G)
    def _elem(e):
      rowb = ch + 2 * e
      for i in range(PER_ELEM // L):
        cols = rh + i * (L // 2)
        m16 = plsc.load_gather(mu_v, [rowb, cols])
        l16 = plsc.load_gather(ls_v, [rowb, cols])
        s = pl.ds(i * L, L)
        out_v[e, s] = m16 + jnp.exp(l16) * eps_v[e, s]

    pltpu.sync_copy(out_v, out_e_hbm.at[pl.ds(g0, G)])


@jax.jit
def _run(mu3d, ls3d, cmu, cls, fids, epsc, epse_flat):
  tmu = _transpose(mu3d)
  tls = _transpose(ls3d)
  mesh = plsc.VectorSubcoreMesh(core_axis_name="c", subcore_axis_name="s")
  return pl.kernel(
      _body,
      out_type=(
          jax.ShapeDtypeStruct((B, PER_ELEM), jnp.float32),
          jax.ShapeDtypeStruct((B,), jnp.float32),
      ),
      mesh=mesh,
      compiler_params=pltpu.CompilerParams(needs_layout_passes=False),
      scratch_types=[
          pltpu.VMEM((E,), jnp.int32),
          pltpu.VMEM((E,), jnp.float32),
          pltpu.VMEM((E,), jnp.float32),
          pltpu.VMEM((E,), jnp.float32),
          pltpu.VMEM((E,), jnp.float32),
          pltpu.VMEM((2 * G,), jnp.int32),
          pltpu.VMEM((2 * G, U), jnp.float32),
          pltpu.VMEM((2 * G, U), jnp.float32),
          pltpu.VMEM((G, PER_ELEM), jnp.float32),
          pltpu.VMEM((G, PER_ELEM), jnp.float32),
          pltpu.SemaphoreType.DMA,
      ],
  )(tmu, tls, cmu, cls, fids, epsc, epse_flat)


def kernel(interaction_mu, interaction_log_sigma, conc_mu, conc_log_sigma,
           family_ids, eps_conc, eps_energy, batch_size):
  epse2d = eps_energy.reshape(B, U * 2)
  fids = family_ids.astype(jnp.int32)
  mu2d = interaction_mu.reshape(U, F * 2)
  ls2d = interaction_log_sigma.reshape(U, F * 2)
  out_e, out_c = _run(mu2d, ls2d, conc_mu, conc_log_sigma, fids,
                      eps_conc, epse2d)
  return out_e.reshape(B, U, 2), out_c, family_ids


# speedup vs baseline: 57.9986x; 1.1748x over previous
import jax
import jax.numpy as jnp
from jax import lax
from jax.experimental import pallas as pl
from jax.experimental.pallas import tpu as pltpu
from jax.experimental.pallas import tpu_sc as plsc

U = 128
F = 100000
B = 16384
NC, NS, L = 2, 16, 16
NW = NC * NS
E = B // NW
G = 64
NCHUNK = E // G
W = G * U * 2
PER_ELEM = U * 2
TB = 8320
LN10 = 2.302585092994046


def _tbody(x_ref, o_ref):
  o_ref[...] = x_ref[...].T


def _transpose(x2d):
  return pl.pallas_call(
      _tbody,
      grid=(pl.cdiv(2 * F, TB),),
      in_specs=[pl.BlockSpec((U, TB), lambda k: (0, k))],
      out_specs=pl.BlockSpec((TB, U), lambda k: (k, 0)),
      out_shape=jax.ShapeDtypeStruct((2 * F, U), jnp.float32),
  )(x2d)


def _body(tmu_hbm, tls_hbm, cmu_hbm, cls_hbm, fids_hbm, epsc_hbm, epse_hbm,
          out_e_hbm, out_c_hbm,
          fids_v, epsc_v, cmu_v, cls_v, outc_v,
          idx_v, mu_v, ls_v, eps_v, out_v, sem):
  wid = lax.axis_index("s") * NC + lax.axis_index("c")
  base = wid * E

  pltpu.sync_copy(fids_hbm.at[pl.ds(base, E)], fids_v)
  pltpu.sync_copy(epsc_hbm.at[pl.ds(base, E)], epsc_v)

  handles = []
  for c in range(E // 128):
    s = pl.ds(c * 128, 128)
    handles.append(pltpu.async_copy(cmu_hbm.at[fids_v.at[s]], cmu_v.at[s], sem))
    handles.append(pltpu.async_copy(cls_hbm.at[fids_v.at[s]], cls_v.at[s], sem))
  for h in handles:
    h.wait()

  @pl.loop(0, E // L)
  def _conc(i):
    s = pl.ds(i * L, L)
    v = cmu_v[s] + jnp.exp(cls_v[s]) * epsc_v[s]
    outc_v[s] = jnp.exp(LN10 * v)

  pltpu.sync_copy(outc_v, out_c_hbm.at[pl.ds(base, E)])

  io2 = lax.iota(jnp.int32, L) * 2
  rh = lax.iota(jnp.int32, L) >> 1
  ch = lax.iota(jnp.int32, L) & 1

  @pl.loop(0, NCHUNK)
  def _chunk(t):
    e0 = t * G
    g0 = base + e0
    pltpu.sync_copy(epse_hbm.at[pl.ds(g0, G)], eps_v)
    for j in range(G // L):
      fvec = fids_v[pl.ds(e0 + j * L, L)]
      f2 = fvec + fvec
      o = 2 * j * L
      plsc.store_scatter(idx_v, [io2 + o], f2)
      plsc.store_scatter(idx_v, [io2 + (o + 1)], f2 + 1)
    h1 = pltpu.async_copy(tmu_hbm.at[idx_v], mu_v, sem)
    h2 = pltpu.async_copy(tls_hbm.at[idx_v], ls_v, sem)
    h1.wait()
    h2.wait()

    @pl.loop(0, G)
    def _elem(e):
      rowb = ch + 2 * e
      for i in range(PER_ELEM // L):
        cols = rh + i * (L // 2)
        m16 = plsc.load_gather(mu_v, [rowb, cols])
        l16 = plsc.load_gather(ls_v, [rowb, cols])
        s = pl.ds(i * L, L)
        out_v[e, s] = m16 + jnp.exp(l16) * eps_v[e, s]

    pltpu.sync_copy(out_v, out_e_hbm.at[pl.ds(g0, G)])


@jax.jit
def _run(mu3d, ls3d, cmu, cls, fids, epsc, epse_flat):
  tmu = _transpose(mu3d)
  tls = _transpose(ls3d)
  mesh = plsc.VectorSubcoreMesh(core_axis_name="c", subcore_axis_name="s")
  return pl.kernel(
      _body,
      out_type=(
          jax.ShapeDtypeStruct((B, PER_ELEM), jnp.float32),
          jax.ShapeDtypeStruct((B,), jnp.float32),
      ),
      mesh=mesh,
      compiler_params=pltpu.CompilerParams(needs_layout_passes=False),
      scratch_types=[
          pltpu.VMEM((E,), jnp.int32),
          pltpu.VMEM((E,), jnp.float32),
          pltpu.VMEM((E,), jnp.float32),
          pltpu.VMEM((E,), jnp.float32),
          pltpu.VMEM((E,), jnp.float32),
          pltpu.VMEM((2 * G,), jnp.int32),
          pltpu.VMEM((2 * G, U), jnp.float32),
          pltpu.VMEM((2 * G, U), jnp.float32),
          pltpu.VMEM((G, PER_ELEM), jnp.float32),
          pltpu.VMEM((G, PER_ELEM), jnp.float32),
          pltpu.SemaphoreType.DMA,
      ],
  )(tmu, tls, cmu, cls, fids, epsc, epse_flat)


def kernel(interaction_mu, interaction_log_sigma, conc_mu, conc_log_sigma,
           family_ids, eps_conc, eps_energy, batch_size):
  epse2d = eps_energy.reshape(B, U * 2)
  fids = family_ids.astype(jnp.int32)
  mu2d = interaction_mu.reshape(U, F * 2)
  ls2d = interaction_log_sigma.reshape(U, F * 2)
  out_e, out_c = _run(mu2d, ls2d, conc_mu, conc_log_sigma, fids,
                      eps_conc, epse2d)
  return out_e.reshape(B, U, 2), out_c, family_ids


# speedup vs baseline: 60.5115x; 1.0433x over previous
import jax
import jax.numpy as jnp
from jax import lax
from jax.experimental import pallas as pl
from jax.experimental.pallas import tpu as pltpu
from jax.experimental.pallas import tpu_sc as plsc

U = 128
F = 100000
B = 16384
NC, NS, L = 2, 16, 16
NW = NC * NS
E = B // NW
G = 64
NCHUNK = E // G
W = G * U * 2
PER_ELEM = U * 2
TB = 8320
LN10 = 2.302585092994046


def _tbody(x_ref, o_ref):
  o_ref[...] = x_ref[...].T


def _transpose(x2d):
  return pl.pallas_call(
      _tbody,
      grid=(pl.cdiv(2 * F, TB),),
      in_specs=[pl.BlockSpec((U, TB), lambda k: (0, k))],
      out_specs=pl.BlockSpec((TB, U), lambda k: (k, 0)),
      out_shape=jax.ShapeDtypeStruct((2 * F, U), jnp.float32),
  )(x2d)


def _body(tmu_hbm, tls_hbm, cmu_hbm, cls_hbm, fids_hbm, epsc_hbm, epse_hbm,
          out_e_hbm, out_c_hbm,
          fids_v, epsc_v, cmu_v, cls_v, outc_v,
          idx0, idx1, mu0, mu1, ls0, ls1, eps0, eps1, out_v,
          sem, sem0, sem1):
  wid = lax.axis_index("s") * NC + lax.axis_index("c")
  base = wid * E

  pltpu.sync_copy(fids_hbm.at[pl.ds(base, E)], fids_v)
  pltpu.sync_copy(epsc_hbm.at[pl.ds(base, E)], epsc_v)

  handles = []
  for c in range(E // 128):
    s = pl.ds(c * 128, 128)
    handles.append(pltpu.async_copy(cmu_hbm.at[fids_v.at[s]], cmu_v.at[s], sem))
    handles.append(pltpu.async_copy(cls_hbm.at[fids_v.at[s]], cls_v.at[s], sem))
  for h in handles:
    h.wait()

  @pl.loop(0, E // L)
  def _conc(i):
    s = pl.ds(i * L, L)
    v = cmu_v[s] + jnp.exp(cls_v[s]) * epsc_v[s]
    outc_v[s] = jnp.exp(LN10 * v)

  pltpu.sync_copy(outc_v, out_c_hbm.at[pl.ds(base, E)])

  io2 = lax.iota(jnp.int32, L) * 2
  rh = lax.iota(jnp.int32, L) >> 1
  ch = lax.iota(jnp.int32, L) & 1

  def _fire(t, idx_b, mu_b, ls_b, eps_b, sem_b):
    e0 = t * G
    g0 = base + e0
    for j in range(G // L):
      fvec = fids_v[pl.ds(e0 + j * L, L)]
      f2 = fvec + fvec
      o = 2 * j * L
      plsc.store_scatter(idx_b, [io2 + o], f2)
      plsc.store_scatter(idx_b, [io2 + (o + 1)], f2 + 1)
    pltpu.async_copy(epse_hbm.at[pl.ds(g0, G)], eps_b, sem_b)
    pltpu.async_copy(tmu_hbm.at[idx_b], mu_b, sem_b)
    pltpu.async_copy(tls_hbm.at[idx_b], ls_b, sem_b)

  bufs = ((idx0, mu0, ls0, eps0, sem0), (idx1, mu1, ls1, eps1, sem1))
  _fire(0, *bufs[0])

  @pl.loop(0, NCHUNK // 2)
  def _t2(t2):
    for p in range(2):
      t = t2 * 2 + p
      idx_b, mu_b, ls_b, eps_b, sem_b = bufs[p]
      nxt = bufs[1 - p]

      @pl.when(t + 1 < NCHUNK)
      def _():
        _fire(t + 1, *nxt)

      g0 = base + t * G
      pltpu.make_async_copy(epse_hbm.at[pl.ds(g0, G)], eps_b, sem_b).wait()
      pltpu.make_async_copy(tmu_hbm.at[idx_b], mu_b, sem_b).wait()
      pltpu.make_async_copy(tls_hbm.at[idx_b], ls_b, sem_b).wait()

      @pl.loop(0, G)
      def _elem(e):
        rowb = ch + 2 * e
        for i in range(PER_ELEM // L):
          cols = rh + i * (L // 2)
          m16 = plsc.load_gather(mu_b, [rowb, cols])
          l16 = plsc.load_gather(ls_b, [rowb, cols])
          s = pl.ds(i * L, L)
          out_v[e, s] = m16 + jnp.exp(l16) * eps_b[e, s]

      pltpu.sync_copy(out_v, out_e_hbm.at[pl.ds(g0, G)])


@jax.jit
def _run(mu3d, ls3d, cmu, cls, fids, epsc, epse_flat):
  tmu = _transpose(mu3d)
  tls = _transpose(ls3d)
  mesh = plsc.VectorSubcoreMesh(core_axis_name="c", subcore_axis_name="s")
  return pl.kernel(
      _body,
      out_type=(
          jax.ShapeDtypeStruct((B, PER_ELEM), jnp.float32),
          jax.ShapeDtypeStruct((B,), jnp.float32),
      ),
      mesh=mesh,
      compiler_params=pltpu.CompilerParams(needs_layout_passes=False),
      scratch_types=[
          pltpu.VMEM((E,), jnp.int32),
          pltpu.VMEM((E,), jnp.float32),
          pltpu.VMEM((E,), jnp.float32),
          pltpu.VMEM((E,), jnp.float32),
          pltpu.VMEM((E,), jnp.float32),
          pltpu.VMEM((2 * G,), jnp.int32),
          pltpu.VMEM((2 * G,), jnp.int32),
          pltpu.VMEM((2 * G, U), jnp.float32),
          pltpu.VMEM((2 * G, U), jnp.float32),
          pltpu.VMEM((2 * G, U), jnp.float32),
          pltpu.VMEM((2 * G, U), jnp.float32),
          pltpu.VMEM((G, PER_ELEM), jnp.float32),
          pltpu.VMEM((G, PER_ELEM), jnp.float32),
          pltpu.VMEM((G, PER_ELEM), jnp.float32),
          pltpu.SemaphoreType.DMA,
          pltpu.SemaphoreType.DMA,
          pltpu.SemaphoreType.DMA,
      ],
  )(tmu, tls, cmu, cls, fids, epsc, epse_flat)


def kernel(interaction_mu, interaction_log_sigma, conc_mu, conc_log_sigma,
           family_ids, eps_conc, eps_energy, batch_size):
  epse2d = eps_energy.reshape(B, U * 2)
  fids = family_ids.astype(jnp.int32)
  mu2d = interaction_mu.reshape(U, F * 2)
  ls2d = interaction_log_sigma.reshape(U, F * 2)
  out_e, out_c = _run(mu2d, ls2d, conc_mu, conc_log_sigma, fids,
                      eps_conc, epse2d)
  return out_e.reshape(B, U, 2), out_c, family_ids


# speedup vs baseline: 60.8399x; 1.0054x over previous
import jax
import jax.numpy as jnp
from jax import lax
from jax.experimental import pallas as pl
from jax.experimental.pallas import tpu as pltpu
from jax.experimental.pallas import tpu_sc as plsc

U = 128
F = 100000
B = 16384
NC, NS, L = 2, 16, 16
NW = NC * NS
E = B // NW
G = 64
NCHUNK = E // G
W = G * U * 2
PER_ELEM = U * 2
TB = 16640
LN10 = 2.302585092994046


def _tbody(x_ref, o_ref):
  o_ref[...] = x_ref[...].T


def _transpose(x2d):
  return pl.pallas_call(
      _tbody,
      grid=(pl.cdiv(2 * F, TB),),
      in_specs=[pl.BlockSpec((U, TB), lambda k: (0, k))],
      out_specs=pl.BlockSpec((TB, U), lambda k: (k, 0)),
      out_shape=jax.ShapeDtypeStruct((2 * F, U), jnp.float32),
  )(x2d)


def _body(tmu_hbm, tls_hbm, cmu_hbm, cls_hbm, fids_hbm, epsc_hbm, epse_hbm,
          out_e_hbm, out_c_hbm,
          fids_v, epsc_v, cmu_v, cls_v, outc_v,
          idx0, idx1, mu0, mu1, ls0, ls1, eps0, eps1, out_v,
          sem, sem0, sem1):
  wid = lax.axis_index("s") * NC + lax.axis_index("c")
  base = wid * E

  pltpu.sync_copy(fids_hbm.at[pl.ds(base, E)], fids_v)
  pltpu.sync_copy(epsc_hbm.at[pl.ds(base, E)], epsc_v)

  handles = []
  for c in range(E // 128):
    s = pl.ds(c * 128, 128)
    handles.append(pltpu.async_copy(cmu_hbm.at[fids_v.at[s]], cmu_v.at[s], sem))
    handles.append(pltpu.async_copy(cls_hbm.at[fids_v.at[s]], cls_v.at[s], sem))
  for h in handles:
    h.wait()

  @pl.loop(0, E // L)
  def _conc(i):
    s = pl.ds(i * L, L)
    v = cmu_v[s] + jnp.exp(cls_v[s]) * epsc_v[s]
    outc_v[s] = jnp.exp(LN10 * v)

  pltpu.sync_copy(outc_v, out_c_hbm.at[pl.ds(base, E)])

  io2 = lax.iota(jnp.int32, L) * 2
  rh = lax.iota(jnp.int32, L) >> 1
  ch = lax.iota(jnp.int32, L) & 1

  def _fire(t, idx_b, mu_b, ls_b, eps_b, sem_b):
    e0 = t * G
    g0 = base + e0
    for j in range(G // L):
      fvec = fids_v[pl.ds(e0 + j * L, L)]
      f2 = fvec + fvec
      o = 2 * j * L
      plsc.store_scatter(idx_b, [io2 + o], f2)
      plsc.store_scatter(idx_b, [io2 + (o + 1)], f2 + 1)
    pltpu.async_copy(epse_hbm.at[pl.ds(g0, G)], eps_b, sem_b)
    pltpu.async_copy(tmu_hbm.at[idx_b], mu_b, sem_b)
    pltpu.async_copy(tls_hbm.at[idx_b], ls_b, sem_b)

  bufs = ((idx0, mu0, ls0, eps0, sem0), (idx1, mu1, ls1, eps1, sem1))
  _fire(0, *bufs[0])

  @pl.loop(0, NCHUNK // 2)
  def _t2(t2):
    for p in range(2):
      t = t2 * 2 + p
      idx_b, mu_b, ls_b, eps_b, sem_b = bufs[p]
      nxt = bufs[1 - p]

      @pl.when(t + 1 < NCHUNK)
      def _():
        _fire(t + 1, *nxt)

      g0 = base + t * G
      pltpu.make_async_copy(epse_hbm.at[pl.ds(g0, G)], eps_b, sem_b).wait()
      pltpu.make_async_copy(tmu_hbm.at[idx_b], mu_b, sem_b).wait()
      pltpu.make_async_copy(tls_hbm.at[idx_b], ls_b, sem_b).wait()

      @pl.loop(0, G)
      def _elem(e):
        rowb = ch + 2 * e
        for i in range(PER_ELEM // L):
          cols = rh + i * (L // 2)
          m16 = plsc.load_gather(mu_b, [rowb, cols])
          l16 = plsc.load_gather(ls_b, [rowb, cols])
          s = pl.ds(i * L, L)
          out_v[e, s] = m16 + jnp.exp(l16) * eps_b[e, s]

      pltpu.sync_copy(out_v, out_e_hbm.at[pl.ds(g0, G)])


@jax.jit
def _run(mu3d, ls3d, cmu, cls, fids, epsc, epse_flat):
  tmu = _transpose(mu3d)
  tls = _transpose(ls3d)
  mesh = plsc.VectorSubcoreMesh(core_axis_name="c", subcore_axis_name="s")
  return pl.kernel(
      _body,
      out_type=(
          jax.ShapeDtypeStruct((B, PER_ELEM), jnp.float32),
          jax.ShapeDtypeStruct((B,), jnp.float32),
      ),
      mesh=mesh,
      compiler_params=pltpu.CompilerParams(needs_layout_passes=False),
      scratch_types=[
          pltpu.VMEM((E,), jnp.int32),
          pltpu.VMEM((E,), jnp.float32),
          pltpu.VMEM((E,), jnp.float32),
          pltpu.VMEM((E,), jnp.float32),
          pltpu.VMEM((E,), jnp.float32),
          pltpu.VMEM((2 * G,), jnp.int32),
          pltpu.VMEM((2 * G,), jnp.int32),
          pltpu.VMEM((2 * G, U), jnp.float32),
          pltpu.VMEM((2 * G, U), jnp.float32),
          pltpu.VMEM((2 * G, U), jnp.float32),
          pltpu.VMEM((2 * G, U), jnp.float32),
          pltpu.VMEM((G, PER_ELEM), jnp.float32),
          pltpu.VMEM((G, PER_ELEM), jnp.float32),
          pltpu.VMEM((G, PER_ELEM), jnp.float32),
          pltpu.SemaphoreType.DMA,
          pltpu.SemaphoreType.DMA,
          pltpu.SemaphoreType.DMA,
      ],
  )(tmu, tls, cmu, cls, fids, epsc, epse_flat)


def kernel(interaction_mu, interaction_log_sigma, conc_mu, conc_log_sigma,
           family_ids, eps_conc, eps_energy, batch_size):
  epse2d = eps_energy.reshape(B, U * 2)
  fids = family_ids.astype(jnp.int32)
  mu2d = interaction_mu.reshape(U, F * 2)
  ls2d = interaction_log_sigma.reshape(U, F * 2)
  out_e, out_c = _run(mu2d, ls2d, conc_mu, conc_log_sigma, fids,
                      eps_conc, epse2d)
  return out_e.reshape(B, U, 2), out_c, family_ids
